# baseline (speedup 1.0000x reference)
import functools

import jax
import jax.numpy as jnp
from jax import lax
from jax.experimental import pallas as pl
from jax.experimental.pallas import tpu as pltpu
from jax.experimental.pallas import tpu_sc as plsc

_N = 10000
_E = 320000
_D = 128
_NC = 2
_NS = 16
_NW = _NC * _NS
_K = 50
_CT = _E // _K
_CW = _CT // _NW
_CG = 40
_NG = _CW // _CG
_NB = 5
_RPT = 624
_RTAIL = _N - _NS * _RPT


def _sc_body(h_hbm, src_hbm, dst_hbm, out_hbm, acc, src_v, dst_v, rows_v,
             sem0, sem1, sem2, sem3, sem4):
    sems = (sem0, sem1, sem2, sem3, sem4)
    cid = lax.axis_index("c")
    sid = lax.axis_index("s")
    w = sid * _NC + cid
    cw0 = w * _CW

    r0 = sid * _RPT
    init_cp = pltpu.async_copy(h_hbm.at[pl.ds(r0, _RPT)],
                               acc.at[pl.ds(r0, _RPT)], sem0)

    @pl.when(sid == _NS - 1)
    def _():
        t0 = _NS * _RPT
        pltpu.sync_copy(h_hbm.at[pl.ds(t0, _RTAIL)], acc.at[pl.ds(t0, _RTAIL)])

    init_cp.wait()
    plsc.subcore_barrier()

    def _gather(k, b):
        return pltpu.async_copy(h_hbm.at[src_v.at[k]], rows_v.at[b], sems[b])

    def _drain(k, b):
        pltpu.make_async_copy(h_hbm.at[src_v.at[k]], rows_v.at[b],
                              sems[b]).wait()
        pltpu.sync_copy(rows_v.at[b], acc.at[dst_v.at[k]], add=True)

    for q in range(_NG):
        pltpu.sync_copy(src_hbm.at[pl.ds(cw0 + q * _CG, _CG)], src_v)
        pltpu.sync_copy(dst_hbm.at[pl.ds(cw0 + q * _CG, _CG)], dst_v)

        for b in range(_NB):
            _gather(b, b)

        def body(j, carry):
            for b in range(_NB):
                k = j * _NB + b
                _drain(k, b)
                _gather(k + _NB, b)
            return carry

        lax.fori_loop(0, (_CG - _NB) // _NB, body, 0)

        for b in range(_NB):
            _drain(_CG - _NB + b, b)

    plsc.subcore_barrier()

    pltpu.sync_copy(acc.at[pl.ds(r0, _RPT)], out_hbm.at[cid, pl.ds(r0, _RPT)])

    @pl.when(sid == _NS - 1)
    def _():
        t0 = _NS * _RPT
        pltpu.sync_copy(acc.at[pl.ds(t0, _RTAIL)],
                        out_hbm.at[cid, pl.ds(t0, _RTAIL)])


_sc_segsum = pl.kernel(
    _sc_body,
    out_type=jax.ShapeDtypeStruct((_NC, _N, _D), jnp.float32),
    mesh=plsc.VectorSubcoreMesh(core_axis_name="c", subcore_axis_name="s"),
    scratch_types=[
        pltpu.VMEM_SHARED((_N, _D), jnp.float32),
        pltpu.VMEM((_CG, _K), jnp.int32),
        pltpu.VMEM((_CG, _K), jnp.int32),
        pltpu.VMEM((_NB, _K, _D), jnp.float32),
        pltpu.SemaphoreType.DMA,
        pltpu.SemaphoreType.DMA,
        pltpu.SemaphoreType.DMA,
        pltpu.SemaphoreType.DMA,
        pltpu.SemaphoreType.DMA,
    ],
)


_BN = 2000


def _mlp_body(acc0, acc1, h, w1, b1, w2, b2, out, *, last):
    t = acc0[0] + acc1[0] - h[...]
    t = jnp.dot(t, w1[...]) + b1[...]
    t = jnp.maximum(t, 0.0)
    t = jnp.dot(t, w2[...]) + b2[...]
    if not last:
        t = jnp.maximum(t, 0.0)
    out[...] = t


def _mlp(acc, h, w1, b1, w2, b2, last):
    row = lambda i: (i, 0)
    full = lambda i: (0, 0)
    return pl.pallas_call(
        functools.partial(_mlp_body, last=last),
        grid=(_N // _BN,),
        in_specs=[
            pl.BlockSpec((1, _BN, _D), lambda i: (0, i, 0)),
            pl.BlockSpec((1, _BN, _D), lambda i: (1, i, 0)),
            pl.BlockSpec((_BN, _D), row),
            pl.BlockSpec((_D, _D), full),
            pl.BlockSpec((1, _D), full),
            pl.BlockSpec((_D, _D), full),
            pl.BlockSpec((1, _D), full),
        ],
        out_specs=pl.BlockSpec((_BN, _D), row),
        out_shape=jax.ShapeDtypeStruct((_N, _D), jnp.float32),
    )(acc, acc, h, w1, b1, w2, b2)


def kernel(x, edge_index, w1_0, b1_0, w2_0, b2_0, w1_1, b1_1, w2_1, b2_1,
           w1_2, b1_2, w2_2, b2_2, w1_3, b1_3, w2_3, b2_3):
    src = edge_index[0].reshape(_CT, _K)
    dst = edge_index[1].reshape(_CT, _K)
    params = [(w1_0, b1_0, w2_0, b2_0), (w1_1, b1_1, w2_1, b2_1),
              (w1_2, b1_2, w2_2, b2_2), (w1_3, b1_3, w2_3, b2_3)]
    h = x
    for l, (w1, b1, w2, b2) in enumerate(params):
        acc = _sc_segsum(h, src, dst)
        h = _mlp(acc, h, w1, b1.reshape(1, _D), w2, b2.reshape(1, _D),
                 last=(l == len(params) - 1))
    return h


# speedup vs baseline: 1.0660x; 1.0660x over previous
import functools

import jax
import jax.numpy as jnp
from jax import lax
from jax.experimental import pallas as pl
from jax.experimental.pallas import tpu as pltpu
from jax.experimental.pallas import tpu_sc as plsc

_N = 10000
_E = 320000
_D = 128
_NC = 2
_NS = 16
_NW = _NC * _NS
_K = 64
_CT = _E // _K
_NLO = 15
_CLO = 152
_CHI = 160
_CG = 40
_NG = 4
_SMAX = _CHI * _K
_NB = 4
_RPT = 624
_RTAIL = _N - _NS * _RPT


def _sc_body(h_hbm, src_hbm, dst_hbm, out_hbm, acc, src_v, dst_v, rows_v,
             sem0, sem1, sem2, sem3):
    sems = (sem0, sem1, sem2, sem3)
    cid = lax.axis_index("c")
    sid = lax.axis_index("s")
    w = sid * _NC + cid
    lo = w < _NLO
    cw0 = jnp.where(lo, _CLO * w, _CHI * w - (_CHI - _CLO) * _NLO)
    ncw = jnp.where(lo, _CLO, _CHI)

    r0 = sid * _RPT
    init_cp = pltpu.async_copy(h_hbm.at[pl.ds(r0, _RPT)],
                               acc.at[pl.ds(r0, _RPT)], sem0)

    @pl.when(sid == _NS - 1)
    def _():
        t0 = _NS * _RPT
        pltpu.sync_copy(h_hbm.at[pl.ds(t0, _RTAIL)], acc.at[pl.ds(t0, _RTAIL)])

    pltpu.sync_copy(src_hbm.at[pl.ds(cw0 * _K, _SMAX)], src_v)

    init_cp.wait()
    plsc.subcore_barrier()

    def _gather(i, b):
        return pltpu.async_copy(h_hbm.at[src_v.at[pl.ds(i * _K, _K)]],
                                rows_v.at[b], sems[b])

    def _drain(i, b, dloc):
        pltpu.make_async_copy(h_hbm.at[src_v.at[pl.ds(i * _K, _K)]],
                              rows_v.at[b], sems[b]).wait()
        pltpu.sync_copy(rows_v.at[b], acc.at[dst_v.at[dloc]], add=True)

    for q in range(_NG):
        pltpu.sync_copy(dst_hbm.at[pl.ds(cw0 + q * _CG, _CG)], dst_v)
        i0 = q * _CG
        gsz = jnp.minimum(ncw - i0, _CG)

        for b in range(_NB):
            _gather(i0 + b, b)

        def body(j, carry):
            for b in range(_NB):
                k = j * _NB + b
                _drain(i0 + k, b, k)
                _gather(i0 + k + _NB, b)
            return carry

        lax.fori_loop(0, (gsz - _NB) // _NB, body, 0)

        for b in range(_NB):
            k = gsz - _NB + b
            _drain(i0 + k, b, k)

    plsc.subcore_barrier()

    pltpu.sync_copy(acc.at[pl.ds(r0, _RPT)], out_hbm.at[cid, pl.ds(r0, _RPT)])

    @pl.when(sid == _NS - 1)
    def _():
        t0 = _NS * _RPT
        pltpu.sync_copy(acc.at[pl.ds(t0, _RTAIL)],
                        out_hbm.at[cid, pl.ds(t0, _RTAIL)])


_sc_segsum = pl.kernel(
    _sc_body,
    out_type=jax.ShapeDtypeStruct((_NC, _N, _D), jnp.float32),
    mesh=plsc.VectorSubcoreMesh(core_axis_name="c", subcore_axis_name="s"),
    scratch_types=[
        pltpu.VMEM_SHARED((_N, _D), jnp.float32),
        pltpu.VMEM((_SMAX,), jnp.int32),
        pltpu.VMEM((_CG, _K), jnp.int32),
        pltpu.VMEM((_NB, _K, _D), jnp.float32),
        pltpu.SemaphoreType.DMA,
        pltpu.SemaphoreType.DMA,
        pltpu.SemaphoreType.DMA,
        pltpu.SemaphoreType.DMA,
    ],
)


_BN = 2000


def _mlp_body(acc0, acc1, h, w1, b1, w2, b2, out, *, last):
    t = acc0[0] + acc1[0] - h[...]
    t = jnp.dot(t, w1[...]) + b1[...]
    t = jnp.maximum(t, 0.0)
    t = jnp.dot(t, w2[...]) + b2[...]
    if not last:
        t = jnp.maximum(t, 0.0)
    out[...] = t


def _mlp(acc, h, w1, b1, w2, b2, last):
    row = lambda i: (i, 0)
    full = lambda i: (0, 0)
    return pl.pallas_call(
        functools.partial(_mlp_body, last=last),
        grid=(_N // _BN,),
        in_specs=[
            pl.BlockSpec((1, _BN, _D), lambda i: (0, i, 0)),
            pl.BlockSpec((1, _BN, _D), lambda i: (1, i, 0)),
            pl.BlockSpec((_BN, _D), row),
            pl.BlockSpec((_D, _D), full),
            pl.BlockSpec((1, _D), full),
            pl.BlockSpec((_D, _D), full),
            pl.BlockSpec((1, _D), full),
        ],
        out_specs=pl.BlockSpec((_BN, _D), row),
        out_shape=jax.ShapeDtypeStruct((_N, _D), jnp.float32),
    )(acc, acc, h, w1, b1, w2, b2)


def kernel(x, edge_index, w1_0, b1_0, w2_0, b2_0, w1_1, b1_1, w2_1, b2_1,
           w1_2, b1_2, w2_2, b2_2, w1_3, b1_3, w2_3, b2_3):
    src = edge_index[0]
    dst = edge_index[1].reshape(_CT, _K)
    params = [(w1_0, b1_0, w2_0, b2_0), (w1_1, b1_1, w2_1, b2_1),
              (w1_2, b1_2, w2_2, b2_2), (w1_3, b1_3, w2_3, b2_3)]
    h = x
    for l, (w1, b1, w2, b2) in enumerate(params):
        acc = _sc_segsum(h, src, dst)
        h = _mlp(acc, h, w1, b1.reshape(1, _D), w2, b2.reshape(1, _D),
                 last=(l == len(params) - 1))
    return h
